# baseline (speedup 1.0000x reference)
import jax
import jax.numpy as jnp
from jax.experimental import pallas as pl
from jax.experimental.pallas import tpu as pltpu


def _mm_kernel(a_ref, b_ref, o_ref):
    o_ref[...] = jnp.dot(a_ref[...].astype(jnp.bfloat16),
                         b_ref[...],
                         preferred_element_type=jnp.float32)


def _propagate(adj, s, bm=512):
    m, k = adj.shape
    n = s.shape[1]
    return pl.pallas_call(
        _mm_kernel,
        grid=(m // bm,),
        in_specs=[
            pl.BlockSpec((bm, k), lambda i: (i, 0)),
            pl.BlockSpec((k, n), lambda i: (0, 0)),
        ],
        out_specs=pl.BlockSpec((bm, n), lambda i: (i, 0)),
        out_shape=jax.ShapeDtypeStruct((m, n), jnp.float32),
        compiler_params=pltpu.CompilerParams(
            dimension_semantics=("parallel",)),
    )(adj, s.astype(jnp.bfloat16))


def kernel(seq, adj, propa_times):
    return jax.lax.fori_loop(
        0, propa_times, lambda _, s: _propagate(adj, s), seq
    )


# speedup vs baseline: 1.3125x; 1.3125x over previous
import jax
import jax.numpy as jnp
from jax.experimental import pallas as pl
from jax.experimental.pallas import tpu as pltpu

_BM = 512


def _fused_kernel(adj_ref, seq_ref, o_ref, adjbf_ref, t_ref):
    p = pl.program_id(0)
    i = pl.program_id(1)

    @pl.when(p == 0)
    def _():
        a = adj_ref[...].astype(jnp.bfloat16)
        adjbf_ref[pl.ds(i * _BM, _BM), :] = a
        t = jnp.dot(a, seq_ref[...],
                    preferred_element_type=jnp.float32)
        t_ref[pl.ds(i * _BM, _BM), :] = t.astype(jnp.bfloat16)

    @pl.when(p == 1)
    def _():
        a = adjbf_ref[pl.ds(i * _BM, _BM), :]
        o_ref[...] = jnp.dot(a, t_ref[...],
                             preferred_element_type=jnp.float32)


def _fused_two_steps(adj, seq):
    m, k = adj.shape
    n = seq.shape[1]
    nblk = m // _BM
    seq = seq.astype(jnp.bfloat16)
    return pl.pallas_call(
        _fused_kernel,
        grid=(2, nblk),
        in_specs=[
            pl.BlockSpec((_BM, k),
                         lambda p, i: (jnp.where(p == 0, i, nblk - 1), 0)),
            pl.BlockSpec((k, n), lambda p, i: (0, 0)),
        ],
        out_specs=pl.BlockSpec((_BM, n),
                               lambda p, i: (jnp.where(p == 1, i, 0), 0)),
        out_shape=jax.ShapeDtypeStruct((m, n), jnp.float32),
        scratch_shapes=[
            pltpu.VMEM((m, k), jnp.bfloat16),
            pltpu.VMEM((m, n), jnp.bfloat16),
        ],
        compiler_params=pltpu.CompilerParams(
            vmem_limit_bytes=100 * 1024 * 1024),
    )(adj, seq)


def _mm_kernel(a_ref, b_ref, o_ref):
    o_ref[...] = jnp.dot(a_ref[...].astype(jnp.bfloat16),
                         b_ref[...].astype(jnp.bfloat16),
                         preferred_element_type=jnp.float32)


def _propagate(adj, s, bm=512):
    m, k = adj.shape
    n = s.shape[1]
    return pl.pallas_call(
        _mm_kernel,
        grid=(m // bm,),
        in_specs=[
            pl.BlockSpec((bm, k), lambda i: (i, 0)),
            pl.BlockSpec((k, n), lambda i: (0, 0)),
        ],
        out_specs=pl.BlockSpec((bm, n), lambda i: (i, 0)),
        out_shape=jax.ShapeDtypeStruct((m, n), jnp.float32),
    )(adj, s)


def kernel(seq, adj, propa_times):
    return jax.lax.cond(
        propa_times == 2,
        lambda: _fused_two_steps(adj, seq),
        lambda: jax.lax.fori_loop(
            0, propa_times, lambda _, s: _propagate(adj, s), seq),
    )
